# baseline (speedup 1.0000x reference)
import functools

import jax
import jax.numpy as jnp
from jax import lax
from jax.experimental import pallas as pl
from jax.experimental.pallas import tpu as pltpu
from jax.experimental.pallas import tpu_sc as plsc

N = 10000
E = 320000
D = 128
C = 40
CP = 64
EPS = 1e-5
NC, NS = 2, 16
NW = NC * NS
K = 80
NCH = 128
EWP = NCH * K
NCHX = NCH + 2
EWPX = NCHX * K
NPAD = 10240
RPT = NPAD // NS
ZR = 80
DW_DEG = 16

_MESH = plsc.VectorSubcoreMesh(core_axis_name="c", subcore_axis_name="s")
_SC_PARAMS = pltpu.CompilerParams(use_tc_tiling_on_sc=False)


@functools.partial(
    pl.kernel,
    out_type=jax.ShapeDtypeStruct((NC, NPAD, DW_DEG), jnp.float32),
    mesh=_MESH,
    scratch_types=[
        pltpu.VMEM_SHARED((NPAD, DW_DEG), jnp.float32),
        pltpu.VMEM((NCH, 2, K), jnp.int32),
        pltpu.VMEM((K, DW_DEG), jnp.float32),
        pltpu.VMEM((ZR, DW_DEG), jnp.float32),
        pltpu.SemaphoreType.DMA,
    ],
    compiler_params=_SC_PARAMS,
)
def _sc_deg(e3_hbm, ones_hbm, zeros_hbm, out_hbm, acc, didx, ones_v, zbuf,
            sem):
    c = lax.axis_index("c")
    s = lax.axis_index("s")
    wid = c * NS + s
    pltpu.sync_copy(zeros_hbm, zbuf)
    pltpu.sync_copy(ones_hbm, ones_v)

    def zb(j, carry):
        pltpu.sync_copy(zbuf, acc.at[pl.ds(s * RPT + j * ZR, ZR)])
        return carry

    lax.fori_loop(0, RPT // ZR, zb, 0)
    pltpu.sync_copy(e3_hbm.at[wid], didx)
    plsc.subcore_barrier()

    def fire(i, carry):
        pltpu.async_copy(ones_v, acc.at[didx.at[i, 1]], sem, add=True)
        return carry

    lax.fori_loop(0, NCH, fire, 0)

    def drain(i, carry):
        pltpu.make_async_copy(ones_v, acc.at[didx.at[i, 1]], sem).wait()
        return carry

    lax.fori_loop(0, NCH, drain, 0)
    plsc.subcore_barrier()

    def outb(j, carry):
        row0 = s * RPT + j * ZR
        pltpu.sync_copy(acc.at[pl.ds(row0, ZR)], zbuf)
        pltpu.sync_copy(zbuf, out_hbm.at[c, pl.ds(row0, ZR)])
        return carry

    lax.fori_loop(0, RPT // ZR, outb, 0)


def _make_sc_agg(dw):
    @functools.partial(
        pl.kernel,
        out_type=jax.ShapeDtypeStruct((NC, NPAD, dw), jnp.float32),
        mesh=_MESH,
        scratch_types=[
            pltpu.VMEM_SHARED((NPAD, dw), jnp.float32),
            pltpu.VMEM((K,), jnp.int32),
            pltpu.VMEM((K,), jnp.int32),
            pltpu.VMEM((K, dw), jnp.float32),
            pltpu.SemaphoreType.DMA,
        ],
        compiler_params=_SC_PARAMS,
    )
    def agg(xs_hbm, src_hbm, dst_hbm, zeros_hbm, out_hbm,
            acc, sidx, didx, msg, sem):
        c = lax.axis_index("c")
        s = lax.axis_index("s")
        wid = c * NS + s
        pltpu.sync_copy(zeros_hbm, msg)

        def zb(j, carry):
            pltpu.sync_copy(msg, acc.at[pl.ds(s * RPT + j * ZR, ZR)])
            return carry

        lax.fori_loop(0, RPT // ZR, zb, 0)
        plsc.subcore_barrier()

        eoff = wid * EWPX

        def body(i, carry):
            base = eoff + i * K
            pltpu.sync_copy(src_hbm.at[pl.ds(base, K)], sidx)
            pltpu.sync_copy(dst_hbm.at[pl.ds(base, K)], didx)
            pltpu.async_copy(xs_hbm.at[sidx], msg, sem).wait()
            pltpu.sync_copy(msg, acc.at[didx], add=True)
            return carry

        lax.fori_loop(0, NCH, body, 0)
        plsc.subcore_barrier()

        def outb(j, carry):
            row0 = s * RPT + j * ZR
            pltpu.sync_copy(acc.at[pl.ds(row0, ZR)], msg)
            pltpu.sync_copy(msg, out_hbm.at[c, pl.ds(row0, ZR)])
            return carry

        lax.fori_loop(0, RPT // ZR, outb, 0)

    return agg


_sc_agg_d = _make_sc_agg(D)
_sc_agg_c = _make_sc_agg(CP)


def _tc_pre_body(x_ref, w1_ref, dp_ref, xs1_ref, dinv_ref):
    deg = dp_ref[0, :N, 0:1] + dp_ref[1, :N, 0:1] + 1.0
    dinv = lax.rsqrt(deg)
    xw = jnp.dot(x_ref[...], w1_ref[...], preferred_element_type=jnp.float32)
    xs1_ref[...] = xw * dinv
    dinv_ref[...] = dinv


def _tc_mid_body(s1_ref, xs1_ref, dinv_ref, b1_ref, g1_ref, be1_ref, w2_ref,
                 xs2_ref):
    dinv = dinv_ref[...]
    t = dinv * (s1_ref[0, :N, :] + s1_ref[1, :N, :] + xs1_ref[...]) + b1_ref[...]
    mean = jnp.mean(t, axis=0, keepdims=True)
    ctr = t - mean
    var = jnp.mean(ctr * ctr, axis=0, keepdims=True)
    h = g1_ref[...] * ctr * lax.rsqrt(var + EPS) + be1_ref[...]
    h = jnp.maximum(h, 0.0)
    xw2 = jnp.dot(h, w2_ref[...], preferred_element_type=jnp.float32)
    xs2_ref[...] = xw2 * dinv


def _tc_final_body(s2_ref, xs2_ref, dinv_ref, b2_ref, g2_ref, be2_ref, o_ref):
    dinv = dinv_ref[...]
    t = dinv * (s2_ref[0, :N, :] + s2_ref[1, :N, :] + xs2_ref[...]) + b2_ref[...]
    mean = jnp.mean(t, axis=0, keepdims=True)
    ctr = t - mean
    var = jnp.mean(ctr * ctr, axis=0, keepdims=True)
    o_ref[...] = g2_ref[...] * ctr * lax.rsqrt(var + EPS) + be2_ref[...]


_tc_pre = pl.pallas_call(
    _tc_pre_body,
    out_shape=[
        jax.ShapeDtypeStruct((N, D), jnp.float32),
        jax.ShapeDtypeStruct((N, 1), jnp.float32),
    ],
)

_tc_mid = pl.pallas_call(
    _tc_mid_body,
    out_shape=jax.ShapeDtypeStruct((N, CP), jnp.float32),
)

_tc_final = pl.pallas_call(
    _tc_final_body,
    out_shape=jax.ShapeDtypeStruct((N, CP), jnp.float32),
)


def kernel(x, edge_index, W1, b1, gamma1, beta1, W2, b2, gamma2, beta2):
    src = edge_index[0].astype(jnp.int32)
    dst = edge_index[1].astype(jnp.int32)
    ew = E // NW
    src_p = jnp.pad(src.reshape(NW, ew), ((0, 0), (0, EWP - ew)),
                    constant_values=0)
    dst_p = jnp.pad(dst.reshape(NW, ew), ((0, 0), (0, EWP - ew)),
                    constant_values=N)
    e3 = jnp.stack([src_p, dst_p], axis=1)
    e3 = e3.reshape(NW, 2, NCH, K).transpose(0, 2, 1, 3)
    src_f = jnp.pad(src_p, ((0, 0), (0, EWPX - EWP))).reshape(-1)
    dst_f = jnp.pad(dst_p, ((0, 0), (0, EWPX - EWP)),
                    constant_values=N).reshape(-1)

    ones16 = jnp.ones((K, DW_DEG), jnp.float32)
    zeros16 = jnp.zeros((ZR, DW_DEG), jnp.float32)
    dp = _sc_deg(e3, ones16, zeros16)

    xs1, dinv = _tc_pre(x, W1, dp)

    zeros_d = jnp.zeros((ZR, D), jnp.float32)
    s1 = _sc_agg_d(xs1, src_f, dst_f, zeros_d)

    W2p = jnp.pad(W2, ((0, 0), (0, CP - C)))
    xs2 = _tc_mid(s1, xs1, dinv, b1[None, :], gamma1[None, :],
                  beta1[None, :], W2p)

    zeros_c = jnp.zeros((ZR, CP), jnp.float32)
    s2 = _sc_agg_c(xs2, src_f, dst_f, zeros_c)

    b2p = jnp.pad(b2, (0, CP - C))[None, :]
    g2p = jnp.pad(gamma2, (0, CP - C))[None, :]
    be2p = jnp.pad(beta2, (0, CP - C))[None, :]
    out = _tc_final(s2, xs2, dinv, b2p, g2p, be2p)
    return out[:, :C]


# speedup vs baseline: 1.0036x; 1.0036x over previous
import functools

import jax
import jax.numpy as jnp
from jax import lax
from jax.experimental import pallas as pl
from jax.experimental.pallas import tpu as pltpu
from jax.experimental.pallas import tpu_sc as plsc

N = 10000
E = 320000
D = 128
C = 40
CP = 64
EPS = 1e-5
NC, NS = 2, 16
NW = NC * NS
K = 80
NCH = 128
EWP = NCH * K
NCHX = NCH + 2
EWPX = NCHX * K
NPAD = 10240
RPT = NPAD // NS
ZR = 80
DW_DEG = 16

_MESH = plsc.VectorSubcoreMesh(core_axis_name="c", subcore_axis_name="s")
_SC_PARAMS = pltpu.CompilerParams(use_tc_tiling_on_sc=False)


@functools.partial(
    pl.kernel,
    out_type=jax.ShapeDtypeStruct((NC, NPAD, DW_DEG), jnp.float32),
    mesh=_MESH,
    scratch_types=[
        pltpu.VMEM_SHARED((NPAD, DW_DEG), jnp.float32),
        pltpu.VMEM((NCH, 2, K), jnp.int32),
        pltpu.VMEM((K, DW_DEG), jnp.float32),
        pltpu.VMEM((ZR, DW_DEG), jnp.float32),
        pltpu.SemaphoreType.DMA,
    ],
    compiler_params=_SC_PARAMS,
)
def _sc_deg(e3_hbm, ones_hbm, zeros_hbm, out_hbm, acc, didx, ones_v, zbuf,
            sem):
    c = lax.axis_index("c")
    s = lax.axis_index("s")
    wid = c * NS + s
    pltpu.sync_copy(zeros_hbm, zbuf)
    pltpu.sync_copy(ones_hbm, ones_v)

    def zb(j, carry):
        pltpu.sync_copy(zbuf, acc.at[pl.ds(s * RPT + j * ZR, ZR)])
        return carry

    lax.fori_loop(0, RPT // ZR, zb, 0)
    pltpu.sync_copy(e3_hbm.at[wid], didx)
    plsc.subcore_barrier()

    def fire(i, carry):
        pltpu.async_copy(ones_v, acc.at[didx.at[i, 1]], sem, add=True)
        return carry

    lax.fori_loop(0, NCH, fire, 0)

    def drain(i, carry):
        pltpu.make_async_copy(ones_v, acc.at[didx.at[i, 1]], sem).wait()
        return carry

    lax.fori_loop(0, NCH, drain, 0)
    plsc.subcore_barrier()

    def outb(j, carry):
        row0 = s * RPT + j * ZR
        pltpu.sync_copy(acc.at[pl.ds(row0, ZR)], zbuf)
        pltpu.sync_copy(zbuf, out_hbm.at[c, pl.ds(row0, ZR)])
        return carry

    lax.fori_loop(0, RPT // ZR, outb, 0)


def _make_sc_agg(dw):
    @functools.partial(
        pl.kernel,
        out_type=jax.ShapeDtypeStruct((NC, NPAD, dw), jnp.float32),
        mesh=_MESH,
        scratch_types=[
            pltpu.VMEM_SHARED((NPAD, dw), jnp.float32),
            pltpu.VMEM((K,), jnp.int32),
            pltpu.VMEM((K,), jnp.int32),
            pltpu.VMEM((K, dw), jnp.float32),
            pltpu.SemaphoreType.DMA,
        ],
        compiler_params=_SC_PARAMS,
    )
    def agg(xs_hbm, src_hbm, dst_hbm, zeros_hbm, out_hbm,
            acc, sidx, didx, msg, sem):
        c = lax.axis_index("c")
        s = lax.axis_index("s")
        wid = c * NS + s
        pltpu.sync_copy(zeros_hbm, msg)

        def zb(j, carry):
            pltpu.sync_copy(msg, acc.at[pl.ds(s * RPT + j * ZR, ZR)])
            return carry

        lax.fori_loop(0, RPT // ZR, zb, 0)
        plsc.subcore_barrier()

        eoff = wid * EWPX

        def body(i, carry):
            base = eoff + i * K
            pltpu.sync_copy(src_hbm.at[pl.ds(base, K)], sidx)
            pltpu.sync_copy(dst_hbm.at[pl.ds(base, K)], didx)
            pltpu.async_copy(xs_hbm.at[sidx], msg, sem).wait()
            pltpu.sync_copy(msg, acc.at[didx], add=True)
            return carry

        lax.fori_loop(0, NCH, body, 0)
        plsc.subcore_barrier()

        def outb(j, carry):
            row0 = s * RPT + j * ZR
            pltpu.sync_copy(acc.at[pl.ds(row0, ZR)], msg)
            pltpu.sync_copy(msg, out_hbm.at[c, pl.ds(row0, ZR)])
            return carry

        lax.fori_loop(0, RPT // ZR, outb, 0)

    return agg


_sc_agg_d = _make_sc_agg(D)
_sc_agg_c = _make_sc_agg(CP)


def _tc_pre_body(x_ref, w1_ref, dp_ref, xs1_ref, dinv_ref):
    deg = dp_ref[0, :N, 0:1] + dp_ref[1, :N, 0:1] + 1.0
    dinv = lax.rsqrt(deg)
    xw = jnp.dot(x_ref[...], w1_ref[...], preferred_element_type=jnp.float32)
    xs1_ref[...] = xw * dinv
    dinv_ref[...] = dinv


def _tc_mid_body(s1_ref, xs1_ref, dinv_ref, b1_ref, g1_ref, be1_ref, w2_ref,
                 xs2_ref):
    dinv = dinv_ref[...]
    t = dinv * (s1_ref[0, :N, :] + s1_ref[1, :N, :] + xs1_ref[...]) + b1_ref[...]
    mean = jnp.mean(t, axis=0, keepdims=True)
    ctr = t - mean
    var = jnp.mean(ctr * ctr, axis=0, keepdims=True)
    h = g1_ref[...] * ctr * lax.rsqrt(var + EPS) + be1_ref[...]
    h = jnp.maximum(h, 0.0)
    xw2 = jnp.dot(h, w2_ref[...], preferred_element_type=jnp.float32)
    xs2_ref[...] = xw2 * dinv


def _tc_final_body(s2_ref, xs2_ref, dinv_ref, b2_ref, g2_ref, be2_ref, o_ref):
    dinv = dinv_ref[...]
    t = dinv * (s2_ref[0, :N, :] + s2_ref[1, :N, :] + xs2_ref[...]) + b2_ref[...]
    mean = jnp.mean(t, axis=0, keepdims=True)
    ctr = t - mean
    var = jnp.mean(ctr * ctr, axis=0, keepdims=True)
    o_ref[...] = g2_ref[...] * ctr * lax.rsqrt(var + EPS) + be2_ref[...]


_tc_pre = pl.pallas_call(
    _tc_pre_body,
    out_shape=[
        jax.ShapeDtypeStruct((N, D), jnp.float32),
        jax.ShapeDtypeStruct((N, 1), jnp.float32),
    ],
)

_tc_mid = pl.pallas_call(
    _tc_mid_body,
    out_shape=jax.ShapeDtypeStruct((N, CP), jnp.float32),
)

_tc_final = pl.pallas_call(
    _tc_final_body,
    out_shape=jax.ShapeDtypeStruct((N, CP), jnp.float32),
)


def kernel(x, edge_index, W1, b1, gamma1, beta1, W2, b2, gamma2, beta2):
    src = edge_index[0].astype(jnp.int32)
    dst = edge_index[1].astype(jnp.int32)
    ew = E // NW
    src_p = jnp.pad(src.reshape(NW, ew), ((0, 0), (0, EWP - ew)),
                    constant_values=0)
    pad_rows = N + jnp.arange(EWP - ew, dtype=jnp.int32)
    dst_p = jnp.concatenate(
        [dst.reshape(NW, ew),
         jnp.broadcast_to(pad_rows, (NW, EWP - ew))], axis=1)
    e3 = jnp.stack([src_p, dst_p], axis=1)
    e3 = e3.reshape(NW, 2, NCH, K).transpose(0, 2, 1, 3)
    src_f = jnp.pad(src_p, ((0, 0), (0, EWPX - EWP))).reshape(-1)
    dst_f = jnp.pad(dst_p, ((0, 0), (0, EWPX - EWP)),
                    constant_values=N).reshape(-1)

    ones16 = jnp.ones((K, DW_DEG), jnp.float32)
    zeros16 = jnp.zeros((ZR, DW_DEG), jnp.float32)
    dp = _sc_deg(e3, ones16, zeros16)

    xs1, dinv = _tc_pre(x, W1, dp)

    zeros_d = jnp.zeros((ZR, D), jnp.float32)
    s1 = _sc_agg_d(xs1, src_f, dst_f, zeros_d)

    W2p = jnp.pad(W2, ((0, 0), (0, CP - C)))
    xs2 = _tc_mid(s1, xs1, dinv, b1[None, :], gamma1[None, :],
                  beta1[None, :], W2p)

    zeros_c = jnp.zeros((ZR, CP), jnp.float32)
    s2 = _sc_agg_c(xs2, src_f, dst_f, zeros_c)

    b2p = jnp.pad(b2, (0, CP - C))[None, :]
    g2p = jnp.pad(gamma2, (0, CP - C))[None, :]
    be2p = jnp.pad(beta2, (0, CP - C))[None, :]
    out = _tc_final(s2, xs2, dinv, b2p, g2p, be2p)
    return out[:, :C]


# speedup vs baseline: 1.5157x; 1.5102x over previous
import functools

import jax
import jax.numpy as jnp
from jax import lax
from jax.experimental import pallas as pl
from jax.experimental.pallas import tpu as pltpu
from jax.experimental.pallas import tpu_sc as plsc

N = 10000
E = 320000
D = 128
C = 40
CP = 64
EPS = 1e-5
NC, NS = 2, 16
NW = NC * NS
EW = E // NW
K = 80
NCH = EW // K
NPAD = 10240
RPT = NPAD // NS
DW_DEG = 16

_MESH = plsc.VectorSubcoreMesh(core_axis_name="c", subcore_axis_name="s")
_SC_PARAMS = pltpu.CompilerParams(use_tc_tiling_on_sc=False)


@functools.partial(
    pl.kernel,
    out_type=jax.ShapeDtypeStruct((NC, NPAD, DW_DEG), jnp.float32),
    mesh=_MESH,
    scratch_types=[
        pltpu.VMEM_SHARED((NPAD, DW_DEG), jnp.float32),
        pltpu.VMEM((K,), jnp.int32),
        pltpu.VMEM((K, DW_DEG), jnp.float32),
        pltpu.VMEM((K, DW_DEG), jnp.float32),
    ],
    compiler_params=_SC_PARAMS,
)
def _sc_deg(dst_hbm, ones_hbm, zeros_hbm, out_hbm, acc, didx, ones_v, stage):
    c = lax.axis_index("c")
    s = lax.axis_index("s")
    wid = c * NS + s
    pltpu.sync_copy(zeros_hbm, stage)
    pltpu.sync_copy(ones_hbm, ones_v)

    def zb(j, carry):
        pltpu.sync_copy(stage, acc.at[pl.ds(s * RPT + j * K, K)])
        return carry

    lax.fori_loop(0, RPT // K, zb, 0)
    plsc.subcore_barrier()

    eoff = wid * EW

    def body(i, carry):
        pltpu.sync_copy(dst_hbm.at[pl.ds(eoff + i * K, K)], didx)
        pltpu.sync_copy(ones_v, acc.at[didx], add=True)
        return carry

    lax.fori_loop(0, NCH, body, 0)
    plsc.subcore_barrier()

    def outb(j, carry):
        row0 = s * RPT + j * K
        pltpu.sync_copy(acc.at[pl.ds(row0, K)], stage)
        pltpu.sync_copy(stage, out_hbm.at[c, pl.ds(row0, K)])
        return carry

    lax.fori_loop(0, RPT // K, outb, 0)


def _make_sc_agg(dw):
    @functools.partial(
        pl.kernel,
        out_type=jax.ShapeDtypeStruct((NC, NPAD, dw), jnp.float32),
        mesh=_MESH,
        scratch_types=[
            pltpu.VMEM_SHARED((NPAD, dw), jnp.float32),
            pltpu.VMEM((K,), jnp.int32),
            pltpu.VMEM((K,), jnp.int32),
            pltpu.VMEM((K, dw), jnp.float32),
            pltpu.SemaphoreType.DMA,
        ],
        compiler_params=_SC_PARAMS,
    )
    def agg(xs_hbm, src_hbm, dst_hbm, zeros_hbm, out_hbm,
            acc, sidx, didx, msg, sem):
        c = lax.axis_index("c")
        s = lax.axis_index("s")
        wid = c * NS + s
        pltpu.sync_copy(zeros_hbm, msg)

        def zb(j, carry):
            pltpu.sync_copy(msg, acc.at[pl.ds(s * RPT + j * K, K)])
            return carry

        lax.fori_loop(0, RPT // K, zb, 0)
        plsc.subcore_barrier()

        eoff = wid * EW

        def body(i, carry):
            base = eoff + i * K
            pltpu.sync_copy(src_hbm.at[pl.ds(base, K)], sidx)
            pltpu.sync_copy(dst_hbm.at[pl.ds(base, K)], didx)
            pltpu.async_copy(xs_hbm.at[sidx], msg, sem).wait()
            pltpu.sync_copy(msg, acc.at[didx], add=True)
            return carry

        lax.fori_loop(0, NCH, body, 0)
        plsc.subcore_barrier()

        def outb(j, carry):
            row0 = s * RPT + j * K
            pltpu.sync_copy(acc.at[pl.ds(row0, K)], msg)
            pltpu.sync_copy(msg, out_hbm.at[c, pl.ds(row0, K)])
            return carry

        lax.fori_loop(0, RPT // K, outb, 0)

    return agg


_sc_agg_d = _make_sc_agg(D)
_sc_agg_c = _make_sc_agg(CP)


def _tc_pre_body(x_ref, w1_ref, dp_ref, xs1_ref, dinv_ref):
    deg = dp_ref[0, :N, 0:1] + dp_ref[1, :N, 0:1] + 1.0
    dinv = lax.rsqrt(deg)
    xw = jnp.dot(x_ref[...], w1_ref[...], preferred_element_type=jnp.float32)
    xs1_ref[...] = xw * dinv
    dinv_ref[...] = dinv


def _tc_mid_body(s1_ref, xs1_ref, dinv_ref, b1_ref, g1_ref, be1_ref, w2_ref,
                 xs2_ref):
    dinv = dinv_ref[...]
    t = dinv * (s1_ref[0, :N, :] + s1_ref[1, :N, :] + xs1_ref[...]) + b1_ref[...]
    mean = jnp.mean(t, axis=0, keepdims=True)
    ctr = t - mean
    var = jnp.mean(ctr * ctr, axis=0, keepdims=True)
    h = g1_ref[...] * ctr * lax.rsqrt(var + EPS) + be1_ref[...]
    h = jnp.maximum(h, 0.0)
    xw2 = jnp.dot(h, w2_ref[...], preferred_element_type=jnp.float32)
    xs2_ref[...] = xw2 * dinv


def _tc_final_body(s2_ref, xs2_ref, dinv_ref, b2_ref, g2_ref, be2_ref, o_ref):
    dinv = dinv_ref[...]
    t = dinv * (s2_ref[0, :N, :] + s2_ref[1, :N, :] + xs2_ref[...]) + b2_ref[...]
    mean = jnp.mean(t, axis=0, keepdims=True)
    ctr = t - mean
    var = jnp.mean(ctr * ctr, axis=0, keepdims=True)
    o_ref[...] = g2_ref[...] * ctr * lax.rsqrt(var + EPS) + be2_ref[...]


_tc_pre = pl.pallas_call(
    _tc_pre_body,
    out_shape=[
        jax.ShapeDtypeStruct((N, D), jnp.float32),
        jax.ShapeDtypeStruct((N, 1), jnp.float32),
    ],
)

_tc_mid = pl.pallas_call(
    _tc_mid_body,
    out_shape=jax.ShapeDtypeStruct((N, CP), jnp.float32),
)

_tc_final = pl.pallas_call(
    _tc_final_body,
    out_shape=jax.ShapeDtypeStruct((N, CP), jnp.float32),
)


def kernel(x, edge_index, W1, b1, gamma1, beta1, W2, b2, gamma2, beta2):
    src = edge_index[0].astype(jnp.int32)
    dst = edge_index[1].astype(jnp.int32)

    ones16 = jnp.ones((K, DW_DEG), jnp.float32)
    zeros16 = jnp.zeros((K, DW_DEG), jnp.float32)
    dp = _sc_deg(dst, ones16, zeros16)

    xs1, dinv = _tc_pre(x, W1, dp)

    zeros_d = jnp.zeros((K, D), jnp.float32)
    s1 = _sc_agg_d(xs1, src, dst, zeros_d)

    W2p = jnp.pad(W2, ((0, 0), (0, CP - C)))
    xs2 = _tc_mid(s1, xs1, dinv, b1[None, :], gamma1[None, :],
                  beta1[None, :], W2p)

    zeros_c = jnp.zeros((K, CP), jnp.float32)
    s2 = _sc_agg_c(xs2, src, dst, zeros_c)

    b2p = jnp.pad(b2, (0, CP - C))[None, :]
    g2p = jnp.pad(gamma2, (0, CP - C))[None, :]
    be2p = jnp.pad(beta2, (0, CP - C))[None, :]
    out = _tc_final(s2, xs2, dinv, b2p, g2p, be2p)
    return out[:, :C]


# speedup vs baseline: 1.6484x; 1.0876x over previous
import functools

import jax
import jax.numpy as jnp
from jax import lax
from jax.experimental import pallas as pl
from jax.experimental.pallas import tpu as pltpu
from jax.experimental.pallas import tpu_sc as plsc

N = 10000
E = 320000
D = 128
C = 40
CP = 64
EPS = 1e-5
NC, NS = 2, 16
NW = NC * NS
EW = E // NW
K = 80
NCH = EW // K
NPAD = 10240
RPT = NPAD // NS
DW_DEG = 16

_MESH = plsc.VectorSubcoreMesh(core_axis_name="c", subcore_axis_name="s")
_SC_PARAMS = pltpu.CompilerParams(use_tc_tiling_on_sc=False)


@functools.partial(
    pl.kernel,
    out_type=jax.ShapeDtypeStruct((NC, NPAD, DW_DEG), jnp.float32),
    mesh=_MESH,
    scratch_types=[
        pltpu.VMEM_SHARED((NPAD, DW_DEG), jnp.float32),
        pltpu.VMEM((NCH, K), jnp.int32),
        pltpu.VMEM((K, DW_DEG), jnp.float32),
        pltpu.VMEM((K, DW_DEG), jnp.float32),
        pltpu.SemaphoreType.DMA,
    ],
    compiler_params=_SC_PARAMS,
)
def _sc_deg(dst3_hbm, ones_hbm, zeros_hbm, out_hbm, acc, didx, ones_v, stage,
            sem):
    c = lax.axis_index("c")
    s = lax.axis_index("s")
    wid = c * NS + s
    pltpu.sync_copy(zeros_hbm, stage)
    pltpu.sync_copy(ones_hbm, ones_v)

    def zb(j, carry):
        pltpu.sync_copy(stage, acc.at[pl.ds(s * RPT + j * K, K)])
        return carry

    lax.fori_loop(0, RPT // K, zb, 0)
    pltpu.sync_copy(dst3_hbm.at[wid], didx)
    plsc.subcore_barrier()

    def fire(i, carry):
        pltpu.async_copy(ones_v, acc.at[didx.at[i]], sem, add=True)
        return carry

    lax.fori_loop(0, NCH, fire, 0)

    def drain(i, carry):
        pltpu.make_async_copy(ones_v, acc.at[didx.at[i]], sem).wait()
        return carry

    lax.fori_loop(0, NCH, drain, 0)
    plsc.subcore_barrier()

    def outb(j, carry):
        row0 = s * RPT + j * K
        pltpu.sync_copy(acc.at[pl.ds(row0, K)], stage)
        pltpu.sync_copy(stage, out_hbm.at[c, pl.ds(row0, K)])
        return carry

    lax.fori_loop(0, RPT // K, outb, 0)


def _make_sc_agg(dw):
    @functools.partial(
        pl.kernel,
        out_type=jax.ShapeDtypeStruct((NC, NPAD, dw), jnp.float32),
        mesh=_MESH,
        scratch_types=[
            pltpu.VMEM_SHARED((NPAD, dw), jnp.float32),
            pltpu.VMEM((K,), jnp.int32),
            pltpu.VMEM((K,), jnp.int32),
            pltpu.VMEM((K, dw), jnp.float32),
            pltpu.SemaphoreType.DMA,
        ],
        compiler_params=_SC_PARAMS,
    )
    def agg(xs_hbm, src_hbm, dst_hbm, zeros_hbm, out_hbm,
            acc, sidx, didx, msg, sem):
        c = lax.axis_index("c")
        s = lax.axis_index("s")
        wid = c * NS + s
        pltpu.sync_copy(zeros_hbm, msg)

        def zb(j, carry):
            pltpu.sync_copy(msg, acc.at[pl.ds(s * RPT + j * K, K)])
            return carry

        lax.fori_loop(0, RPT // K, zb, 0)
        plsc.subcore_barrier()

        eoff = wid * EW

        def body(i, carry):
            base = eoff + i * K
            pltpu.sync_copy(src_hbm.at[pl.ds(base, K)], sidx)
            pltpu.sync_copy(dst_hbm.at[pl.ds(base, K)], didx)
            pltpu.async_copy(xs_hbm.at[sidx], msg, sem).wait()
            pltpu.sync_copy(msg, acc.at[didx], add=True)
            return carry

        lax.fori_loop(0, NCH, body, 0)
        plsc.subcore_barrier()

        def outb(j, carry):
            row0 = s * RPT + j * K
            pltpu.sync_copy(acc.at[pl.ds(row0, K)], msg)
            pltpu.sync_copy(msg, out_hbm.at[c, pl.ds(row0, K)])
            return carry

        lax.fori_loop(0, RPT // K, outb, 0)

    return agg


_sc_agg_d = _make_sc_agg(D)
_sc_agg_c = _make_sc_agg(CP)


def _tc_pre_body(x_ref, w1_ref, dp_ref, xs1_ref, dinv_ref):
    deg = dp_ref[0, :N, 0:1] + dp_ref[1, :N, 0:1] + 1.0
    dinv = lax.rsqrt(deg)
    xw = jnp.dot(x_ref[...], w1_ref[...], preferred_element_type=jnp.float32)
    xs1_ref[...] = xw * dinv
    dinv_ref[...] = dinv


def _tc_mid_body(s1_ref, xs1_ref, dinv_ref, b1_ref, g1_ref, be1_ref, w2_ref,
                 xs2_ref):
    dinv = dinv_ref[...]
    t = dinv * (s1_ref[0, :N, :] + s1_ref[1, :N, :] + xs1_ref[...]) + b1_ref[...]
    mean = jnp.mean(t, axis=0, keepdims=True)
    ctr = t - mean
    var = jnp.mean(ctr * ctr, axis=0, keepdims=True)
    h = g1_ref[...] * ctr * lax.rsqrt(var + EPS) + be1_ref[...]
    h = jnp.maximum(h, 0.0)
    xw2 = jnp.dot(h, w2_ref[...], preferred_element_type=jnp.float32)
    xs2_ref[...] = xw2 * dinv


def _tc_final_body(s2_ref, xs2_ref, dinv_ref, b2_ref, g2_ref, be2_ref, o_ref):
    dinv = dinv_ref[...]
    t = dinv * (s2_ref[0, :N, :] + s2_ref[1, :N, :] + xs2_ref[...]) + b2_ref[...]
    mean = jnp.mean(t, axis=0, keepdims=True)
    ctr = t - mean
    var = jnp.mean(ctr * ctr, axis=0, keepdims=True)
    o_ref[...] = g2_ref[...] * ctr * lax.rsqrt(var + EPS) + be2_ref[...]


_tc_pre = pl.pallas_call(
    _tc_pre_body,
    out_shape=[
        jax.ShapeDtypeStruct((N, D), jnp.float32),
        jax.ShapeDtypeStruct((N, 1), jnp.float32),
    ],
)

_tc_mid = pl.pallas_call(
    _tc_mid_body,
    out_shape=jax.ShapeDtypeStruct((N, CP), jnp.float32),
)

_tc_final = pl.pallas_call(
    _tc_final_body,
    out_shape=jax.ShapeDtypeStruct((N, CP), jnp.float32),
)


def kernel(x, edge_index, W1, b1, gamma1, beta1, W2, b2, gamma2, beta2):
    src = edge_index[0].astype(jnp.int32)
    dst = edge_index[1].astype(jnp.int32)

    ones16 = jnp.ones((K, DW_DEG), jnp.float32)
    zeros16 = jnp.zeros((K, DW_DEG), jnp.float32)
    dst3 = dst.reshape(NW, NCH, K)
    dp = _sc_deg(dst3, ones16, zeros16)

    xs1, dinv = _tc_pre(x, W1, dp)

    zeros_d = jnp.zeros((K, D), jnp.float32)
    s1 = _sc_agg_d(xs1, src, dst, zeros_d)

    W2p = jnp.pad(W2, ((0, 0), (0, CP - C)))
    xs2 = _tc_mid(s1, xs1, dinv, b1[None, :], gamma1[None, :],
                  beta1[None, :], W2p)

    zeros_c = jnp.zeros((K, CP), jnp.float32)
    s2 = _sc_agg_c(xs2, src, dst, zeros_c)

    b2p = jnp.pad(b2, (0, CP - C))[None, :]
    g2p = jnp.pad(gamma2, (0, CP - C))[None, :]
    be2p = jnp.pad(beta2, (0, CP - C))[None, :]
    out = _tc_final(s2, xs2, dinv, b2p, g2p, be2p)
    return out[:, :C]


# speedup vs baseline: 2.4502x; 1.4864x over previous
import functools

import jax
import jax.numpy as jnp
from jax import lax
from jax.experimental import pallas as pl
from jax.experimental.pallas import tpu as pltpu
from jax.experimental.pallas import tpu_sc as plsc

N = 10000
E = 320000
D = 128
C = 40
CP = 64
EPS = 1e-5
NC, NS = 2, 16
NW = NC * NS
EW = E // NW
K = 80
NCH = EW // K
NPAD = 10240
RPT = NPAD // NS
DW_DEG = 16

_MESH = plsc.VectorSubcoreMesh(core_axis_name="c", subcore_axis_name="s")
_SC_PARAMS = pltpu.CompilerParams(use_tc_tiling_on_sc=False)


@functools.partial(
    pl.kernel,
    out_type=jax.ShapeDtypeStruct((NC, NPAD, DW_DEG), jnp.float32),
    mesh=_MESH,
    scratch_types=[
        pltpu.VMEM_SHARED((NPAD, DW_DEG), jnp.float32),
        pltpu.VMEM((NCH, K), jnp.int32),
        pltpu.VMEM((K, DW_DEG), jnp.float32),
        pltpu.VMEM((K, DW_DEG), jnp.float32),
        pltpu.SemaphoreType.DMA,
    ],
    compiler_params=_SC_PARAMS,
)
def _sc_deg(dst3_hbm, ones_hbm, zeros_hbm, out_hbm, acc, didx, ones_v, stage,
            sem):
    c = lax.axis_index("c")
    s = lax.axis_index("s")
    wid = c * NS + s
    pltpu.sync_copy(zeros_hbm, stage)
    pltpu.sync_copy(ones_hbm, ones_v)

    def zb(j, carry):
        pltpu.sync_copy(stage, acc.at[pl.ds(s * RPT + j * K, K)])
        return carry

    lax.fori_loop(0, RPT // K, zb, 0)
    pltpu.sync_copy(dst3_hbm.at[wid], didx)
    plsc.subcore_barrier()

    def fire(i, carry):
        pltpu.async_copy(ones_v, acc.at[didx.at[i]], sem, add=True)
        return carry

    lax.fori_loop(0, NCH, fire, 0)

    def drain(i, carry):
        pltpu.make_async_copy(ones_v, acc.at[didx.at[i]], sem).wait()
        return carry

    lax.fori_loop(0, NCH, drain, 0)
    plsc.subcore_barrier()

    def outb(j, carry):
        row0 = s * RPT + j * K
        pltpu.sync_copy(acc.at[pl.ds(row0, K)], stage)
        pltpu.sync_copy(stage, out_hbm.at[c, pl.ds(row0, K)])
        return carry

    lax.fori_loop(0, RPT // K, outb, 0)


def _make_sc_agg(dw):
    @functools.partial(
        pl.kernel,
        out_type=jax.ShapeDtypeStruct((NC, NPAD, dw), jnp.float32),
        mesh=_MESH,
        scratch_types=[
            pltpu.VMEM_SHARED((NPAD, dw), jnp.float32),
            pltpu.VMEM((K,), jnp.int32),
            pltpu.VMEM((K,), jnp.int32),
            pltpu.VMEM((K, dw), jnp.float32),
            pltpu.VMEM((K,), jnp.int32),
            pltpu.VMEM((K,), jnp.int32),
            pltpu.VMEM((K, dw), jnp.float32),
            pltpu.SemaphoreType.DMA,
            pltpu.SemaphoreType.DMA,
        ],
        compiler_params=_SC_PARAMS,
    )
    def agg(xs_hbm, src_hbm, dst_hbm, zeros_hbm, out_hbm,
            acc, sidxa, didxa, msga, sidxb, didxb, msgb, sema, semb):
        c = lax.axis_index("c")
        s = lax.axis_index("s")
        wid = c * NS + s
        pltpu.sync_copy(zeros_hbm, msga)

        def zb(j, carry):
            pltpu.sync_copy(msga, acc.at[pl.ds(s * RPT + j * K, K)])
            return carry

        lax.fori_loop(0, RPT // K, zb, 0)
        plsc.subcore_barrier()

        eoff = wid * EW

        pltpu.sync_copy(src_hbm.at[pl.ds(eoff, K)], sidxa)
        pltpu.sync_copy(dst_hbm.at[pl.ds(eoff, K)], didxa)
        pltpu.async_copy(xs_hbm.at[sidxa], msga, sema)
        pltpu.sync_copy(src_hbm.at[pl.ds(eoff + K, K)], sidxb)
        pltpu.sync_copy(dst_hbm.at[pl.ds(eoff + K, K)], didxb)
        pltpu.async_copy(xs_hbm.at[sidxb], msgb, semb)

        def body(i, carry):
            j0 = 2 * i
            j1 = 2 * i + 1
            pltpu.make_async_copy(xs_hbm.at[sidxa], msga, sema).wait()
            pltpu.sync_copy(msga, acc.at[didxa], add=True)
            base0 = eoff + (j0 + 2) * K
            pltpu.sync_copy(src_hbm.at[pl.ds(base0, K)], sidxa)
            pltpu.sync_copy(dst_hbm.at[pl.ds(base0, K)], didxa)
            pltpu.async_copy(xs_hbm.at[sidxa], msga, sema)

            pltpu.make_async_copy(xs_hbm.at[sidxb], msgb, semb).wait()
            pltpu.sync_copy(msgb, acc.at[didxb], add=True)
            base1 = eoff + (j1 + 2) * K
            pltpu.sync_copy(src_hbm.at[pl.ds(base1, K)], sidxb)
            pltpu.sync_copy(dst_hbm.at[pl.ds(base1, K)], didxb)
            pltpu.async_copy(xs_hbm.at[sidxb], msgb, semb)

            return carry

        lax.fori_loop(0, (NCH - 1) // 2, body, 0)
        pltpu.make_async_copy(xs_hbm.at[sidxa], msga, sema).wait()
        pltpu.sync_copy(msga, acc.at[didxa], add=True)
        base_t = eoff + NCH * K
        pltpu.sync_copy(src_hbm.at[pl.ds(base_t, K)], sidxa)
        pltpu.async_copy(xs_hbm.at[sidxa], msga, sema)
        pltpu.make_async_copy(xs_hbm.at[sidxa], msga, sema).wait()
        pltpu.make_async_copy(xs_hbm.at[sidxb], msgb, semb).wait()
        plsc.subcore_barrier()

        def outb(j, carry):
            row0 = s * RPT + j * K
            pltpu.sync_copy(acc.at[pl.ds(row0, K)], msga)
            pltpu.sync_copy(msga, out_hbm.at[c, pl.ds(row0, K)])
            return carry

        lax.fori_loop(0, RPT // K, outb, 0)

    return agg


_sc_agg_d = _make_sc_agg(D)
_sc_agg_c = _make_sc_agg(CP)


def _tc_pre_body(x_ref, w1_ref, dp_ref, xs1_ref, dinv_ref):
    deg = dp_ref[0, :N, 0:1] + dp_ref[1, :N, 0:1] + 1.0
    dinv = lax.rsqrt(deg)
    xw = jnp.dot(x_ref[...], w1_ref[...], preferred_element_type=jnp.float32)
    xs1_ref[...] = xw * dinv
    dinv_ref[...] = dinv


def _tc_mid_body(s1_ref, xs1_ref, dinv_ref, b1_ref, g1_ref, be1_ref, w2_ref,
                 xs2_ref):
    dinv = dinv_ref[...]
    t = dinv * (s1_ref[0, :N, :] + s1_ref[1, :N, :] + xs1_ref[...]) + b1_ref[...]
    mean = jnp.mean(t, axis=0, keepdims=True)
    ctr = t - mean
    var = jnp.mean(ctr * ctr, axis=0, keepdims=True)
    h = g1_ref[...] * ctr * lax.rsqrt(var + EPS) + be1_ref[...]
    h = jnp.maximum(h, 0.0)
    xw2 = jnp.dot(h, w2_ref[...], preferred_element_type=jnp.float32)
    xs2_ref[...] = xw2 * dinv


def _tc_final_body(s2_ref, xs2_ref, dinv_ref, b2_ref, g2_ref, be2_ref, o_ref):
    dinv = dinv_ref[...]
    t = dinv * (s2_ref[0, :N, :] + s2_ref[1, :N, :] + xs2_ref[...]) + b2_ref[...]
    mean = jnp.mean(t, axis=0, keepdims=True)
    ctr = t - mean
    var = jnp.mean(ctr * ctr, axis=0, keepdims=True)
    o_ref[...] = g2_ref[...] * ctr * lax.rsqrt(var + EPS) + be2_ref[...]


_tc_pre = pl.pallas_call(
    _tc_pre_body,
    out_shape=[
        jax.ShapeDtypeStruct((N, D), jnp.float32),
        jax.ShapeDtypeStruct((N, 1), jnp.float32),
    ],
)

_tc_mid = pl.pallas_call(
    _tc_mid_body,
    out_shape=jax.ShapeDtypeStruct((N, CP), jnp.float32),
)

_tc_final = pl.pallas_call(
    _tc_final_body,
    out_shape=jax.ShapeDtypeStruct((N, CP), jnp.float32),
)


def kernel(x, edge_index, W1, b1, gamma1, beta1, W2, b2, gamma2, beta2):
    src = edge_index[0].astype(jnp.int32)
    dst = edge_index[1].astype(jnp.int32)
    src_x = jnp.pad(src, (0, 2 * K))
    dst_x = jnp.pad(dst, (0, 2 * K))

    ones16 = jnp.ones((K, DW_DEG), jnp.float32)
    zeros16 = jnp.zeros((K, DW_DEG), jnp.float32)
    dst3 = dst.reshape(NW, NCH, K)
    dp = _sc_deg(dst3, ones16, zeros16)

    xs1, dinv = _tc_pre(x, W1, dp)

    zeros_d = jnp.zeros((K, D), jnp.float32)
    s1 = _sc_agg_d(xs1, src_x, dst_x, zeros_d)

    W2p = jnp.pad(W2, ((0, 0), (0, CP - C)))
    xs2 = _tc_mid(s1, xs1, dinv, b1[None, :], gamma1[None, :],
                  beta1[None, :], W2p)

    zeros_c = jnp.zeros((K, CP), jnp.float32)
    s2 = _sc_agg_c(xs2, src_x, dst_x, zeros_c)

    b2p = jnp.pad(b2, (0, CP - C))[None, :]
    g2p = jnp.pad(gamma2, (0, CP - C))[None, :]
    be2p = jnp.pad(beta2, (0, CP - C))[None, :]
    out = _tc_final(s2, xs2, dinv, b2p, g2p, be2p)
    return out[:, :C]
